# initial kernel scaffold (unmeasured)
import jax
import jax.numpy as jnp
from jax import lax
from jax.experimental import pallas as pl
from jax.experimental.pallas import tpu as pltpu

N_DEV = 32


def kernel(A, B):
    m_per, k = A.shape
    _, n = B.shape
    M = N_DEV * m_per

    def body(a_ref, b_ref, out_ref, abuf_ref, send_sems, recv_sems):
        my = lax.axis_index("i")
        left = lax.rem(my + N_DEV - 1, N_DEV)
        right = lax.rem(my + 1, N_DEV)

        barrier_sem = pltpu.get_barrier_semaphore()
        pl.semaphore_signal(barrier_sem, inc=1, device_id=(left,),
                            device_id_type=pl.DeviceIdType.MESH)
        pl.semaphore_signal(barrier_sem, inc=1, device_id=(right,),
                            device_id_type=pl.DeviceIdType.MESH)
        pl.semaphore_wait(barrier_sem, 2)

        abuf_ref[pl.ds(my * m_per, m_per), :] = a_ref[:, :]
        out_ref[pl.ds(my * m_per, m_per), :] = jnp.dot(
            a_ref[:, :], b_ref[:, :], preferred_element_type=jnp.float32)

        for h in range(N_DEV - 1):
            o_send = lax.rem(my - h + N_DEV, N_DEV)
            o_recv = lax.rem(my - h - 1 + N_DEV, N_DEV)
            rdma = pltpu.make_async_remote_copy(
                src_ref=abuf_ref.at[pl.ds(o_send * m_per, m_per)],
                dst_ref=abuf_ref.at[pl.ds(o_send * m_per, m_per)],
                send_sem=send_sems.at[h],
                recv_sem=recv_sems.at[h],
                device_id=(right,),
                device_id_type=pl.DeviceIdType.MESH,
            )
            rdma.start()
            rdma.wait()
            out_ref[pl.ds(o_recv * m_per, m_per), :] = jnp.dot(
                abuf_ref[pl.ds(o_recv * m_per, m_per), :], b_ref[:, :],
                preferred_element_type=jnp.float32)

    return pl.pallas_call(
        body,
        out_shape=jax.ShapeDtypeStruct((M, n), jnp.float32),
        in_specs=[
            pl.BlockSpec(memory_space=pltpu.VMEM),
            pl.BlockSpec(memory_space=pltpu.VMEM),
        ],
        out_specs=pl.BlockSpec(memory_space=pltpu.VMEM),
        scratch_shapes=[
            pltpu.VMEM((M, k), jnp.float32),
            pltpu.SemaphoreType.DMA((N_DEV - 1,)),
            pltpu.SemaphoreType.DMA((N_DEV - 1,)),
        ],
        compiler_params=pltpu.CompilerParams(collective_id=0),
    )(A, B)


# baseline (device time: 302872 ns/iter reference)
import jax
import jax.numpy as jnp
from jax import lax
from jax.experimental import pallas as pl
from jax.experimental.pallas import tpu as pltpu

N_DEV = 32


def kernel(A, B):
    m_per, k = A.shape
    _, n = B.shape
    M = N_DEV * m_per

    def body(a_ref, b_ref, out_ref, abuf_ref, cout_ref, copy_sem,
             send_sems, recv_sems):
        my = lax.axis_index("i")
        left = lax.rem(my + N_DEV - 1, N_DEV)
        right = lax.rem(my + 1, N_DEV)

        barrier_sem = pltpu.get_barrier_semaphore()
        pl.semaphore_signal(barrier_sem, inc=1, device_id=(left,),
                            device_id_type=pl.DeviceIdType.MESH)
        pl.semaphore_signal(barrier_sem, inc=1, device_id=(right,),
                            device_id_type=pl.DeviceIdType.MESH)
        pl.semaphore_wait(barrier_sem, 2)

        def emit_chunk(origin, a_chunk):
            cout_ref[:, :] = jnp.dot(a_chunk, b_ref[:, :],
                                     preferred_element_type=jnp.float32)
            cp = pltpu.make_async_copy(
                cout_ref, out_ref.at[pl.ds(origin * m_per, m_per)], copy_sem)
            cp.start()
            cp.wait()

        abuf_ref[pl.ds(my * m_per, m_per), :] = a_ref[:, :]
        emit_chunk(my, a_ref[:, :])

        for h in range(N_DEV - 1):
            o_send = lax.rem(my - h + N_DEV, N_DEV)
            o_recv = lax.rem(my - h - 1 + N_DEV, N_DEV)
            rdma = pltpu.make_async_remote_copy(
                src_ref=abuf_ref.at[pl.ds(o_send * m_per, m_per)],
                dst_ref=abuf_ref.at[pl.ds(o_send * m_per, m_per)],
                send_sem=send_sems.at[h],
                recv_sem=recv_sems.at[h],
                device_id=(right,),
                device_id_type=pl.DeviceIdType.MESH,
            )
            rdma.start()
            rdma.wait()
            emit_chunk(o_recv, abuf_ref[pl.ds(o_recv * m_per, m_per), :])

    return pl.pallas_call(
        body,
        out_shape=jax.ShapeDtypeStruct((M, n), jnp.float32),
        in_specs=[
            pl.BlockSpec(memory_space=pltpu.VMEM),
            pl.BlockSpec(memory_space=pltpu.VMEM),
        ],
        out_specs=pl.BlockSpec(memory_space=pl.ANY),
        scratch_shapes=[
            pltpu.VMEM((M, k), jnp.float32),
            pltpu.VMEM((m_per, n), jnp.float32),
            pltpu.SemaphoreType.DMA,
            pltpu.SemaphoreType.DMA((N_DEV - 1,)),
            pltpu.SemaphoreType.DMA((N_DEV - 1,)),
        ],
        compiler_params=pltpu.CompilerParams(collective_id=0),
    )(A, B)


# device time: 210869 ns/iter; 1.4363x vs baseline; 1.4363x over previous
import jax
import jax.numpy as jnp
from jax import lax
from jax.experimental import pallas as pl
from jax.experimental.pallas import tpu as pltpu

N_DEV = 32
R_HOPS = N_DEV // 2
L_HOPS = N_DEV - 1 - R_HOPS


def kernel(A, B):
    m_per, k = A.shape
    _, n = B.shape
    M = N_DEV * m_per

    def body(a_ref, b_ref, out_ref, abuf_ref, cout_ref, copy_sems,
             r_send_sems, r_recv_sems, l_send_sems, l_recv_sems):
        my = lax.axis_index("i")
        left = lax.rem(my + N_DEV - 1, N_DEV)
        right = lax.rem(my + 1, N_DEV)

        barrier_sem = pltpu.get_barrier_semaphore()
        pl.semaphore_signal(barrier_sem, inc=1, device_id=(left,),
                            device_id_type=pl.DeviceIdType.MESH)
        pl.semaphore_signal(barrier_sem, inc=1, device_id=(right,),
                            device_id_type=pl.DeviceIdType.MESH)
        pl.semaphore_wait(barrier_sem, 2)

        def a_slot(origin):
            return abuf_ref.at[pl.ds(origin * m_per, m_per)]

        def r_rdma(h):
            o = lax.rem(my - h + N_DEV, N_DEV)
            return pltpu.make_async_remote_copy(
                src_ref=a_slot(o), dst_ref=a_slot(o),
                send_sem=r_send_sems.at[h], recv_sem=r_recv_sems.at[h],
                device_id=(right,), device_id_type=pl.DeviceIdType.MESH)

        def l_rdma(h):
            o = lax.rem(my + h, N_DEV)
            return pltpu.make_async_remote_copy(
                src_ref=a_slot(o), dst_ref=a_slot(o),
                send_sem=l_send_sems.at[h], recv_sem=l_recv_sems.at[h],
                device_id=(left,), device_id_type=pl.DeviceIdType.MESH)

        emit_state = {"idx": 0, "pending": [None, None]}

        def emit_chunk(origin):
            slot = emit_state["idx"] % 2
            emit_state["idx"] += 1
            if emit_state["pending"][slot] is not None:
                emit_state["pending"][slot].wait()
            cout_ref[slot, :, :] = jnp.dot(
                a_slot(origin)[:, :], b_ref[:, :],
                preferred_element_type=jnp.float32)
            cp = pltpu.make_async_copy(
                cout_ref.at[slot],
                out_ref.at[pl.ds(origin * m_per, m_per)],
                copy_sems.at[slot])
            cp.start()
            emit_state["pending"][slot] = cp

        abuf_ref[pl.ds(my * m_per, m_per), :] = a_ref[:, :]
        r_descs = [r_rdma(0)]
        r_descs[0].start()
        l_descs = [l_rdma(0)]
        l_descs[0].start()
        emit_chunk(my)

        for h in range(R_HOPS):
            r_descs[h].wait_recv()
            if h + 1 < R_HOPS:
                d = r_rdma(h + 1)
                d.start()
                r_descs.append(d)
            emit_chunk(lax.rem(my - h - 1 + N_DEV, N_DEV))

            if h < L_HOPS:
                l_descs[h].wait_recv()
                if h + 1 < L_HOPS:
                    d = l_rdma(h + 1)
                    d.start()
                    l_descs.append(d)
                emit_chunk(lax.rem(my + h + 1, N_DEV))

        for d in r_descs:
            d.wait_send()
        for d in l_descs:
            d.wait_send()
        for cp in emit_state["pending"]:
            if cp is not None:
                cp.wait()

    return pl.pallas_call(
        body,
        out_shape=jax.ShapeDtypeStruct((M, n), jnp.float32),
        in_specs=[
            pl.BlockSpec(memory_space=pltpu.VMEM),
            pl.BlockSpec(memory_space=pltpu.VMEM),
        ],
        out_specs=pl.BlockSpec(memory_space=pl.ANY),
        scratch_shapes=[
            pltpu.VMEM((M, k), jnp.float32),
            pltpu.VMEM((2, m_per, n), jnp.float32),
            pltpu.SemaphoreType.DMA((2,)),
            pltpu.SemaphoreType.DMA((R_HOPS,)),
            pltpu.SemaphoreType.DMA((R_HOPS,)),
            pltpu.SemaphoreType.DMA((L_HOPS,)),
            pltpu.SemaphoreType.DMA((L_HOPS,)),
        ],
        compiler_params=pltpu.CompilerParams(collective_id=0),
    )(A, B)


# device time: 122450 ns/iter; 2.4734x vs baseline; 1.7221x over previous
import jax
import jax.numpy as jnp
from jax import lax
from jax.experimental import pallas as pl
from jax.experimental.pallas import tpu as pltpu

N_DEV = 32
R_HOPS = N_DEV // 2
L_HOPS = N_DEV - 1 - R_HOPS


def kernel(A, B):
    m_per, k = A.shape
    _, n = B.shape
    M = N_DEV * m_per

    def body(a_ref, b_ref, out_ref, abuf_ref, bbf_ref, cout_ref, copy_sems,
             r_send_sems, r_recv_sems, l_send_sems, l_recv_sems):
        my = lax.axis_index("i")
        left = lax.rem(my + N_DEV - 1, N_DEV)
        right = lax.rem(my + 1, N_DEV)

        barrier_sem = pltpu.get_barrier_semaphore()
        pl.semaphore_signal(barrier_sem, inc=1, device_id=(left,),
                            device_id_type=pl.DeviceIdType.MESH)
        pl.semaphore_signal(barrier_sem, inc=1, device_id=(right,),
                            device_id_type=pl.DeviceIdType.MESH)
        pl.semaphore_wait(barrier_sem, 2)

        def a_slot(origin):
            return abuf_ref.at[pl.ds(origin * m_per, m_per)]

        def r_rdma(h):
            o = lax.rem(my - h + N_DEV, N_DEV)
            return pltpu.make_async_remote_copy(
                src_ref=a_slot(o), dst_ref=a_slot(o),
                send_sem=r_send_sems.at[h], recv_sem=r_recv_sems.at[h],
                device_id=(right,), device_id_type=pl.DeviceIdType.MESH)

        def l_rdma(h):
            o = lax.rem(my + h, N_DEV)
            return pltpu.make_async_remote_copy(
                src_ref=a_slot(o), dst_ref=a_slot(o),
                send_sem=l_send_sems.at[h], recv_sem=l_recv_sems.at[h],
                device_id=(left,), device_id_type=pl.DeviceIdType.MESH)

        emit_state = {"idx": 0, "pending": [None, None]}

        def emit_chunk(origin):
            slot = emit_state["idx"] % 2
            emit_state["idx"] += 1
            if emit_state["pending"][slot] is not None:
                emit_state["pending"][slot].wait()
            cout_ref[slot, :, :] = jnp.dot(
                a_slot(origin)[:, :], bbf_ref[:, :],
                preferred_element_type=jnp.float32)
            cp = pltpu.make_async_copy(
                cout_ref.at[slot],
                out_ref.at[pl.ds(origin * m_per, m_per)],
                copy_sems.at[slot])
            cp.start()
            emit_state["pending"][slot] = cp

        bbf_ref[:, :] = b_ref[:, :].astype(jnp.bfloat16)
        abuf_ref[pl.ds(my * m_per, m_per), :] = a_ref[:, :].astype(jnp.bfloat16)
        r_descs = [r_rdma(0)]
        r_descs[0].start()
        l_descs = [l_rdma(0)]
        l_descs[0].start()
        emit_chunk(my)

        for h in range(R_HOPS):
            r_descs[h].wait_recv()
            if h + 1 < R_HOPS:
                d = r_rdma(h + 1)
                d.start()
                r_descs.append(d)
            emit_chunk(lax.rem(my - h - 1 + N_DEV, N_DEV))

            if h < L_HOPS:
                l_descs[h].wait_recv()
                if h + 1 < L_HOPS:
                    d = l_rdma(h + 1)
                    d.start()
                    l_descs.append(d)
                emit_chunk(lax.rem(my + h + 1, N_DEV))

        for d in r_descs:
            d.wait_send()
        for d in l_descs:
            d.wait_send()
        for cp in emit_state["pending"]:
            if cp is not None:
                cp.wait()

    return pl.pallas_call(
        body,
        out_shape=jax.ShapeDtypeStruct((M, n), jnp.float32),
        in_specs=[
            pl.BlockSpec(memory_space=pltpu.VMEM),
            pl.BlockSpec(memory_space=pltpu.VMEM),
        ],
        out_specs=pl.BlockSpec(memory_space=pl.ANY),
        scratch_shapes=[
            pltpu.VMEM((M, k), jnp.bfloat16),
            pltpu.VMEM((k, n), jnp.bfloat16),
            pltpu.VMEM((2, m_per, n), jnp.float32),
            pltpu.SemaphoreType.DMA((2,)),
            pltpu.SemaphoreType.DMA((R_HOPS,)),
            pltpu.SemaphoreType.DMA((R_HOPS,)),
            pltpu.SemaphoreType.DMA((L_HOPS,)),
            pltpu.SemaphoreType.DMA((L_HOPS,)),
        ],
        compiler_params=pltpu.CompilerParams(collective_id=0),
    )(A, B)
